# baseline (speedup 1.0000x reference)
import jax
import jax.numpy as jnp
from jax import lax
from jax.experimental import pallas as pl
from jax.experimental.pallas import tpu as pltpu
from jax.experimental.pallas import tpu_sc as plsc

_NC, _NS, _L = 2, 16, 16
_NW = _NC * _NS
_N = 10000
_NP = 10240
_E = 320000
_EPT = _E // _NW
_BB = 80
_NBAT = _EPT // _BB
_CHUNK = _NP // _NS

_vec_mesh_cache = []


def _vec_mesh():
    if not _vec_mesh_cache:
        _vec_mesh_cache.append(plsc.VectorSubcoreMesh(
            core_axis_name="c", subcore_axis_name="s",
            num_cores=_NC, num_subcores=_NS))
    return _vec_mesh_cache[0]

import dataclasses as _dataclasses
_sc_cp = pltpu.CompilerParams()
for _fname, _fval in (("needs_layout_passes", False), ("use_tc_tiling_on_sc", False)):
    if _fname in pltpu.CompilerParams.__dataclass_fields__:
        _sc_cp = _dataclasses.replace(_sc_cp, **{_fname: _fval})


def _sc_esum_body(src_hbm, ew_hbm, out_hbm, srcb, rows, zbuf, acc, sem):
    cid = lax.axis_index("c")
    sid = lax.axis_index("s")
    wid = sid * _NC + cid
    base = wid * _EPT
    z16 = jnp.zeros((_L,), jnp.float32)

    @pl.loop(0, _BB)
    def _(i):
        zbuf[i, :] = z16

    @pl.loop(0, _CHUNK // _BB)
    def _(k):
        pltpu.sync_copy(zbuf, acc.at[pl.ds(sid * _CHUNK + k * _BB, _BB)])

    plsc.subcore_barrier()

    @pl.loop(0, _NBAT)
    def _(b):
        off = base + b * _BB
        pltpu.async_copy(src_hbm.at[pl.ds(off, _BB)], srcb, sem).wait()
        pltpu.async_copy(ew_hbm.at[pl.ds(off, _BB)], rows, sem).wait()
        pltpu.sync_copy(rows, acc.at[srcb], add=True)

    plsc.subcore_barrier()
    pltpu.sync_copy(acc.at[pl.ds(sid * _CHUNK, _CHUNK)],
                    out_hbm.at[cid, pl.ds(sid * _CHUNK, _CHUNK)])


def _sc_esum(src, ewide):
    return pl.kernel(
        _sc_esum_body,
        out_type=jax.ShapeDtypeStruct((_NC, _NP, 16), jnp.float32),
        mesh=_vec_mesh(),
        compiler_params=_sc_cp,
        scratch_types=[
            pltpu.VMEM((_BB,), jnp.int32),
            pltpu.VMEM((_BB, 16), jnp.float32),
            pltpu.VMEM((_BB, 16), jnp.float32),
            pltpu.VMEM_SHARED((_NP, 16), jnp.float32),
            pltpu.SemaphoreType.DMA,
        ],
    )(src, ewide)


def _sc_bc_body(src_hbm, dst_hbm, ew_hbm, y12_hbm, out_hbm,
                srcb, dstb, crows, grows, srows, zbuf, acc, sem, gsem):
    cid = lax.axis_index("c")
    sid = lax.axis_index("s")
    wid = sid * _NC + cid
    base = wid * _EPT
    z16 = jnp.zeros((_L,), jnp.float32)

    @pl.loop(0, _BB)
    def _(i):
        zbuf[i, :] = z16

    @pl.loop(0, _CHUNK // _BB)
    def _(k):
        pltpu.sync_copy(zbuf, acc.at[pl.ds(sid * _CHUNK + k * _BB, _BB)])

    plsc.subcore_barrier()

    @pl.loop(0, _NBAT)
    def _(b):
        off = base + b * _BB
        pltpu.async_copy(src_hbm.at[pl.ds(off, _BB)], srcb, sem).wait()
        pltpu.async_copy(dst_hbm.at[pl.ds(off, _BB)], dstb, sem).wait()
        pltpu.async_copy(ew_hbm.at[pl.ds(off, _BB)], crows, sem).wait()
        pltpu.async_copy(y12_hbm.at[srcb], grows, gsem).wait()

        @pl.loop(0, _BB)
        def _(j):
            srows[j, :] = grows[j, :] * crows[j, :]

        pltpu.sync_copy(srows, acc.at[dstb], add=True)

    plsc.subcore_barrier()
    pltpu.sync_copy(acc.at[pl.ds(sid * _CHUNK, _CHUNK)],
                    out_hbm.at[cid, pl.ds(sid * _CHUNK, _CHUNK)])


def _sc_bc(src, dst, ewide, y12):
    return pl.kernel(
        _sc_bc_body,
        out_type=jax.ShapeDtypeStruct((_NC, _NP, 16), jnp.float32),
        mesh=_vec_mesh(),
        compiler_params=_sc_cp,
        scratch_types=[
            pltpu.VMEM((_BB,), jnp.int32),
            pltpu.VMEM((_BB,), jnp.int32),
            pltpu.VMEM((_BB, 16), jnp.float32),
            pltpu.VMEM((_BB, 16), jnp.float32),
            pltpu.VMEM((_BB, 16), jnp.float32),
            pltpu.VMEM((_BB, 16), jnp.float32),
            pltpu.VMEM_SHARED((_NP, 16), jnp.float32),
            pltpu.SemaphoreType.DMA,
            pltpu.SemaphoreType.DMA,
        ],
    )(src, dst, ewide, y12)


def _sc_a_body(src_hbm, dst_hbm, e_hbm, s_hbm, inv_hbm, out_hbm,
               srcb, dstb, eb, sv, invv, vtmp, rowbuf, zbuf, acc, sem):
    cid = lax.axis_index("c")
    sid = lax.axis_index("s")
    wid = sid * _NC + cid
    base = wid * _EPT
    z16 = jnp.zeros((_L,), jnp.float32)

    @pl.loop(0, _BB)
    def _(i):
        zbuf[i, :] = z16

    @pl.loop(0, _CHUNK // _BB)
    def _(k):
        pltpu.sync_copy(zbuf, acc.at[pl.ds(sid * _CHUNK + k * _BB, _BB)])

    plsc.subcore_barrier()
    pltpu.async_copy(s_hbm, sv, sem).wait()
    pltpu.async_copy(inv_hbm, invv, sem).wait()

    @pl.loop(0, _NBAT)
    def _(b):
        off = base + b * _BB
        pltpu.async_copy(src_hbm.at[pl.ds(off, _BB)], srcb, sem).wait()
        pltpu.async_copy(dst_hbm.at[pl.ds(off, _BB)], dstb, sem).wait()
        pltpu.async_copy(e_hbm.at[pl.ds(off, _BB)], eb, sem).wait()

        @pl.loop(0, _BB, step=_L)
        def _(j0):
            sidx = srcb[pl.ds(j0, _L)]
            v = plsc.load_gather(sv, [sidx]) * plsc.load_gather(invv, [sidx])
            vtmp[...] = v * eb[pl.ds(j0, _L)]
            for jj in range(_L):
                cst = jnp.full((_L,), jj, jnp.int32)
                rowbuf[j0 + jj, :] = plsc.load_gather(vtmp, [cst])

        pltpu.sync_copy(rowbuf, acc.at[dstb], add=True)

    plsc.subcore_barrier()
    pltpu.sync_copy(acc.at[pl.ds(sid * _CHUNK, _CHUNK)],
                    out_hbm.at[cid, pl.ds(sid * _CHUNK, _CHUNK)])


def _sc_a(src, dst, e, s, inv):
    return pl.kernel(
        _sc_a_body,
        out_type=jax.ShapeDtypeStruct((_NC, _NP, 16), jnp.float32),
        mesh=_vec_mesh(),
        compiler_params=_sc_cp,
        scratch_types=[
            pltpu.VMEM((_BB,), jnp.int32),
            pltpu.VMEM((_BB,), jnp.int32),
            pltpu.VMEM((_BB,), jnp.float32),
            pltpu.VMEM((_NP,), jnp.float32),
            pltpu.VMEM((_NP,), jnp.float32),
            pltpu.VMEM((_L,), jnp.float32),
            pltpu.VMEM((_BB, 16), jnp.float32),
            pltpu.VMEM((_BB, 16), jnp.float32),
            pltpu.VMEM_SHARED((_NP, 16), jnp.float32),
            pltpu.SemaphoreType.DMA,
        ],
    )(src, dst, e, s, inv)


def _tc_ew_body(w_ref, b_ref, ea_ref, out_ref):
    r0 = ea_ref[0, :]
    r1 = ea_ref[1, :]
    r2 = ea_ref[2, :]
    r3 = ea_ref[3, :]
    out_ref[0, :] = jnp.exp(r0 * w_ref[0, 0] + r1 * w_ref[1, 0]
                            + r2 * w_ref[2, 0] + r3 * w_ref[3, 0] + b_ref[0])
    out_ref[1, :] = jnp.exp(r0 * w_ref[0, 1] + r1 * w_ref[1, 1]
                            + r2 * w_ref[2, 1] + r3 * w_ref[3, 1] + b_ref[1])


def _tc_ew(eaT, we, be):
    nb = 10
    be_blk = _E // nb
    return pl.pallas_call(
        _tc_ew_body,
        grid=(nb,),
        in_specs=[
            pl.BlockSpec(memory_space=pltpu.SMEM),
            pl.BlockSpec(memory_space=pltpu.SMEM),
            pl.BlockSpec((4, be_blk), lambda i: (0, i)),
        ],
        out_specs=pl.BlockSpec((2, be_blk), lambda i: (0, i)),
        out_shape=jax.ShapeDtypeStruct((2, _E), jnp.float32),
    )(we, be, eaT)


def _tc_y12_body(part_ref, xg_ref, y12_ref, inv_ref):
    xg = xg_ref[...]
    es1 = part_ref[0][:, 0:8] + part_ref[1][:, 0:8]
    es2 = part_ref[0][:, 8:16] + part_ref[1][:, 8:16]
    inv1 = jnp.where(es1 > 0, 1.0 / es1, 0.0)
    inv2 = jnp.where(es2 > 0, 1.0 / es2, 0.0)
    inv_ref[:, 0:1] = inv1[:, 0:1]
    inv_ref[:, 1:2] = inv2[:, 0:1]
    y12_ref[:, 0:8] = xg * inv1
    y12_ref[:, 8:16] = xg * inv2


def _tc_y12(esum_part, xgp):
    return pl.pallas_call(
        _tc_y12_body,
        out_shape=(
            jax.ShapeDtypeStruct((_NP, 16), jnp.float32),
            jax.ShapeDtypeStruct((_NP, 2), jnp.float32),
        ),
    )(esum_part, xgp)


def _final_dense_kernel(g3_ref, w_ref, b_ref, out_ref):
    g3 = g3_ref[...]
    res = g3 @ w_ref[...] + b_ref[...]
    res = jnp.where(jnp.arange(8)[None, :] == 0, jnp.clip(res, -0.1, 1.0), res)
    out_ref[...] = res


_BLK = 1024
_NBLK = _NP // _BLK


def _lstm_update(G, c):
    i = jax.nn.sigmoid(G[:, 0:16])
    f = jax.nn.sigmoid(G[:, 16:32])
    g = jnp.tanh(G[:, 32:48])
    o = jax.nn.sigmoid(G[:, 48:64])
    c2 = f * c + i * g
    h2 = o * jnp.tanh(c2)
    return h2, c2


def _tc_enc_body(x_ref, xg_ref, xseq_ref,
                 w0_ref, b0_ref, w1_ref, b1_ref,
                 wg0_ref, bg0_ref, wg1_ref, bg1_ref,
                 wl1_ref, bl1_ref, wl2_ref, bl2_ref, wl2b_ref, bl2b_ref,
                 s1_ref, hsf_ref, hsb_ref):
    f32 = jnp.float32
    bf16 = jnp.bfloat16
    x = x_ref[...]
    prof = jnp.maximum(x @ w0_ref[...] + b0_ref[...], 0.0) @ w1_ref[...] + b1_ref[...]
    xg = xg_ref[...]
    geo = jnp.maximum(xg @ wg0_ref[...] + bg0_ref[...], 0.0) @ wg1_ref[...] + bg1_ref[...]

    wl1 = wl1_ref[...]
    bl1 = bl1_ref[...]
    B = _BLK
    z = jnp.zeros((B, 16), f32)

    def step1(t, carry):
        hf, cf, hb, cb = carry
        xtf = xseq_ref[t].astype(bf16)
        xtb = xseq_ref[23 - t].astype(bf16)
        A = jnp.concatenate([xtf, hf.astype(bf16), xtb, hb.astype(bf16)], axis=1)
        G = jnp.dot(A, wl1, preferred_element_type=f32) + bl1
        hf, cf = _lstm_update(G[:, 0:64], cf)
        hb, cb = _lstm_update(G[:, 64:128], cb)
        hsf_ref[t] = hf.astype(bf16)
        hsb_ref[23 - t] = hb.astype(bf16)
        return hf, cf, hb, cb

    lax.fori_loop(0, 24, step1, (z, z, z, z))

    wl2 = wl2_ref[...]
    bl2 = bl2_ref[...]

    def step2a(i, carry):
        h2f, c2f, h2b, c2b = carry
        A = jnp.concatenate([hsf_ref[i], hsb_ref[i], h2f.astype(bf16),
                             hsf_ref[23 - i], hsb_ref[23 - i], h2b.astype(bf16)], axis=1)
        G = jnp.dot(A, wl2, preferred_element_type=f32) + bl2
        h2f, c2f = _lstm_update(G[:, 0:64], c2f)
        h2b, c2b = _lstm_update(G[:, 64:128], c2b)
        return h2f, c2f, h2b, c2b

    h2f, _, h2b, c2b = lax.fori_loop(0, 6, step2a, (z, z, z, z))
    t5f = h2f

    wl2b = wl2b_ref[...]
    bl2b = bl2b_ref[...]

    def step2b(i, carry):
        h2b, c2b = carry
        A = jnp.concatenate([hsf_ref[23 - i], hsb_ref[23 - i], h2b.astype(bf16)], axis=1)
        G = jnp.dot(A, wl2b, preferred_element_type=f32) + bl2b
        return _lstm_update(G, c2b)

    t5b, _ = lax.fori_loop(6, 19, step2b, (h2b, c2b))

    s = (jnp.sum(prof, axis=1, keepdims=True)
         + jnp.sum(geo, axis=1, keepdims=True)
         + jnp.sum(t5f, axis=1, keepdims=True)
         + jnp.sum(t5b, axis=1, keepdims=True))
    s1_ref[...] = s


def _tc_encoder(x_p, xgp, xseq, wlist):
    full = lambda shape: pl.BlockSpec(shape, lambda i: tuple(0 for _ in shape))
    in_specs = [
        pl.BlockSpec((_BLK, 128), lambda i: (i, 0)),
        pl.BlockSpec((_BLK, 8), lambda i: (i, 0)),
        pl.BlockSpec((24, _BLK, 8), lambda i: (0, i, 0)),
    ] + [full(w.shape) for w in wlist]
    return pl.pallas_call(
        _tc_enc_body,
        grid=(_NBLK,),
        in_specs=in_specs,
        out_specs=pl.BlockSpec((_BLK, 1), lambda i: (i, 0)),
        out_shape=jax.ShapeDtypeStruct((_NP, 1), jnp.float32),
        scratch_shapes=[
            pltpu.VMEM((24, _BLK, 16), jnp.bfloat16),
            pltpu.VMEM((24, _BLK, 16), jnp.bfloat16),
        ],
    )(x_p, xgp, xseq, *wlist)


def _tc_ubuild_body(co, apart_ref, bcpart_ref, xg_ref, sc_ref, ucat_ref, g_ref, su_ref):
    A8 = apart_ref[0][:, 0:8] + apart_ref[1][:, 0:8]
    BC = bcpart_ref[0][:, co:co + 8] + bcpart_ref[1][:, co:co + 8]
    zcol = jnp.zeros((_NP, 1), jnp.float32)
    U = jnp.concatenate([BC[:, 0:6], A8[:, 0:1], zcol], axis=1)
    xgs = jnp.concatenate([xg_ref[...][:, 0:6], sc_ref[...], zcol], axis=1)
    ucat_ref[...] = jnp.concatenate([U, xgs], axis=1)
    g_ref[...] = jax.lax.dot_general(U, U, (((0,), (0,)), ((), ())),
                                     preferred_element_type=jnp.float32,
                                     precision=jax.lax.Precision.HIGHEST)
    su_ref[...] = jnp.sum(U, axis=0, keepdims=True)


def _tc_ubuild(co, a_part, bc_part, xgp, scol):
    import functools
    return pl.pallas_call(
        functools.partial(_tc_ubuild_body, co),
        out_shape=(
            jax.ShapeDtypeStruct((_NP, 16), jnp.float32),
            jax.ShapeDtypeStruct((8, 8), jnp.float32),
            jax.ShapeDtypeStruct((1, 8), jnp.float32),
        ),
    )(a_part, bc_part, xgp, scol)


def _tc_dense_body(final, ucat_ref, g_ref, su_ref, wcat_ref, m_ref,
                   bng_ref, bnb_ref, wall_ref, ball_ref, out_ref):
    R = jnp.dot(ucat_ref[...], wcat_ref[...],
                preferred_element_type=jnp.float32,
                precision=jax.lax.Precision.HIGHEST)
    out = R[:, 0:32]
    xt = R[:, 32:64]
    M = m_ref[...]
    ninv = 1.0 / _N
    mean = jnp.dot(su_ref[...], M, precision=jax.lax.Precision.HIGHEST) * ninv
    GM = jnp.dot(g_ref[...], M, precision=jax.lax.Precision.HIGHEST)
    e2 = jnp.sum(M * GM, axis=0, keepdims=True) * ninv
    ve = e2 - mean * mean + 1e-5
    r = jax.lax.rsqrt(ve)
    r = r * (1.5 - 0.5 * ve * r * r)
    outn = (out - mean) * r * bng_ref[...] + bnb_ref[...]
    g2 = jnp.maximum(outn, 0.0) + xt
    if final:
        res = jnp.dot(g2, wall_ref[...], precision=jax.lax.Precision.HIGHEST) + ball_ref[...]
        res = jnp.where(jnp.arange(8)[None, :] == 0, jnp.clip(res, -0.1, 1.0), res)
        out_ref[...] = res
    else:
        out_ref[...] = jnp.sum(g2, axis=1, keepdims=True)


def _tc_dense(final, ucat, G, su, wcat, M, bng, bnb, wall, ball):
    import functools
    oshape = (_NP, 8) if final else (_NP, 1)
    return pl.pallas_call(
        functools.partial(_tc_dense_body, final),
        out_shape=jax.ShapeDtypeStruct(oshape, jnp.float32),
    )(ucat, G, su, wcat, M, bng, bnb, wall, ball)


def kernel(x, x_geo, time_series_profile, edge_attr, params, edge_index):
    p = params
    f32 = jnp.float32
    src = edge_index[0].astype(jnp.int32)
    dst = edge_index[1].astype(jnp.int32)

    we = jnp.stack([p['gnn'][0]['edge_w'][0], p['gnn'][1]['edge_w'][0]], axis=1)
    be = jnp.stack([p['gnn'][0]['edge_b'][0], p['gnn'][1]['edge_b'][0]])
    eaT = edge_attr.T

    xgp = jnp.zeros((_NP, 8), f32)
    xgp = xgp.at[:_N, :5].set(x_geo).at[:_N, 5].set(1.0)

    e12 = _tc_ew(eaT, we, be)
    e1 = e12[0]
    e2 = e12[1]
    ewide = jnp.repeat(e12.T, 8, axis=1)

    esum_part = _sc_esum(src, ewide)

    y12, inv12 = _tc_y12(esum_part, xgp)
    inv1 = inv12[:, 0]
    inv2 = inv12[:, 1]

    bc_part = _sc_bc(src, dst, ewide, y12)

    l1, l2 = p['lstm']

    def wcat(lp, in_d):
        W = jnp.zeros((2 * (in_d + 16), 128), f32)
        W = W.at[0:in_d, 0:64].set(lp['wih_f'].T)
        W = W.at[in_d:in_d + 16, 0:64].set(lp['whh_f'].T)
        W = W.at[in_d + 16:2 * in_d + 16, 64:128].set(lp['wih_b'].T)
        W = W.at[2 * in_d + 16:, 64:128].set(lp['whh_b'].T)
        b = jnp.concatenate([lp['bih_f'] + lp['bhh_f'], lp['bih_b'] + lp['bhh_b']])[None]
        return W.astype(jnp.bfloat16), b

    wl1, bl1 = wcat(l1, 8)
    wl2, bl2 = wcat(l2, 32)
    wl2b = jnp.concatenate([l2['wih_b'].T, l2['whh_b'].T], axis=0).astype(jnp.bfloat16)
    bl2b = (l2['bih_b'] + l2['bhh_b'])[None]

    wg0 = jnp.zeros((8, 32), f32).at[:5, :].set(p['geo_w0'].T)
    wlist = [
        p['mlp_w0'].T, p['mlp_b0'][None], p['mlp_w1'].T, p['mlp_b1'][None],
        wg0, p['geo_b0'][None], p['geo_w1'].T, p['geo_b1'][None],
        wl1, bl1, wl2, bl2, wl2b, bl2b,
    ]
    x_p = jnp.pad(x, ((0, _NP - _N), (0, 0)))
    xseq = jnp.pad(jnp.transpose(time_series_profile, (2, 0, 1)),
                   ((0, 0), (0, _NP - _N), (0, 0)))
    s1col = _tc_encoder(x_p, xgp, xseq, wlist)

    def mk_M(gp):
        fsum = gp['feat_w'].sum(1)
        M = jnp.zeros((8, 32), f32)
        M = M.at[0:5, :].set(gp['beta_w'].T)
        M = M.at[5, :].set(gp['beta_b'])
        M = M.at[6, :].set(fsum)
        bwaug = jnp.zeros((8, 32), f32)
        bwaug = bwaug.at[0:5, :].set(gp['beta_w'].T)
        bwaug = bwaug.at[5, :].set(gp['beta_b'])
        bwaug = bwaug.at[6, :].set(fsum)
        wcat = jnp.zeros((16, 64), f32)
        wcat = wcat.at[0:8, 0:32].set(M)
        wcat = wcat.at[8:16, 32:64].set(bwaug)
        return M, wcat, gp['bn_g'][None], gp['bn_b'][None]

    M1, wcat1, bng1, bnb1 = mk_M(p['gnn'][0])
    M2, wcat2, bng2, bnb2 = mk_M(p['gnn'][1])
    w_all = jnp.zeros((32, 8), f32).at[:, 0].set(p['out_w'][0]).at[:, 1:3].set(p['sp_w'].T)
    b_all = jnp.zeros((1, 8), f32).at[0, 0].set(p['out_b'][0]).at[0, 1:3].set(p['sp_b'])

    s1flat = s1col[:, 0]
    a1_part = _sc_a(src, dst, e1, s1flat, inv1)

    ucat1, G1, su1 = _tc_ubuild(0, a1_part, bc_part, xgp, s1col)
    s2col = _tc_dense(False, ucat1, G1, su1, wcat1, M1, bng1, bnb1, w_all, b_all)

    a2_part = _sc_a(src, dst, e2, s2col[:, 0], inv2)

    ucat2, G2, su2 = _tc_ubuild(8, a2_part, bc_part, xgp, s2col)
    res = _tc_dense(True, ucat2, G2, su2, wcat2, M2, bng2, bnb2, w_all, b_all)
    return (res[:_N, 0:1], res[:_N, 1:3])


# speedup vs baseline: 1.3338x; 1.3338x over previous
import jax
import jax.numpy as jnp
from jax import lax
from jax.experimental import pallas as pl
from jax.experimental.pallas import tpu as pltpu
from jax.experimental.pallas import tpu_sc as plsc

_NC, _NS, _L = 2, 16, 16
_NW = _NC * _NS
_N = 10000
_NP = 10240
_E = 320000
_EPT = _E // _NW
_BB = 80
_NBAT = _EPT // _BB
_CHUNK = _NP // _NS

_vec_mesh_cache = []


def _vec_mesh():
    if not _vec_mesh_cache:
        _vec_mesh_cache.append(plsc.VectorSubcoreMesh(
            core_axis_name="c", subcore_axis_name="s",
            num_cores=_NC, num_subcores=_NS))
    return _vec_mesh_cache[0]

import dataclasses as _dataclasses
_sc_cp = pltpu.CompilerParams()
for _fname, _fval in (("needs_layout_passes", False), ("use_tc_tiling_on_sc", False)):
    if _fname in pltpu.CompilerParams.__dataclass_fields__:
        _sc_cp = _dataclasses.replace(_sc_cp, **{_fname: _fval})


def _sc_esum_body(src2_hbm, ew3_hbm, out_hbm, srcv2, rows0, rows1, zbuf, acc, sem, sem0, sem1):
    cid = lax.axis_index("c")
    sid = lax.axis_index("s")
    wid = sid * _NC + cid
    brow = wid * _NBAT
    z16 = jnp.zeros((_L,), jnp.float32)

    @pl.loop(0, _BB)
    def _(i):
        zbuf[i, :] = z16

    @pl.loop(0, _CHUNK // _BB)
    def _(k):
        pltpu.sync_copy(zbuf, acc.at[pl.ds(sid * _CHUNK + k * _BB, _BB)])

    pltpu.async_copy(src2_hbm.at[pl.ds(brow, _NBAT)], srcv2, sem).wait()
    plsc.subcore_barrier()

    @pl.loop(0, _NBAT - 1, step=2)
    def _(b):
        cp0 = pltpu.async_copy(ew3_hbm.at[brow + b], rows0, sem0)
        cp1 = pltpu.async_copy(ew3_hbm.at[brow + b + 1], rows1, sem1)
        cp0.wait()
        pltpu.sync_copy(rows0, acc.at[srcv2.at[b]], add=True)
        cp1.wait()
        pltpu.sync_copy(rows1, acc.at[srcv2.at[b + 1]], add=True)

    pltpu.async_copy(ew3_hbm.at[brow + _NBAT - 1], rows0, sem0).wait()
    pltpu.sync_copy(rows0, acc.at[srcv2.at[_NBAT - 1]], add=True)

    plsc.subcore_barrier()
    pltpu.sync_copy(acc.at[pl.ds(sid * _CHUNK, _CHUNK)],
                    out_hbm.at[cid, pl.ds(sid * _CHUNK, _CHUNK)])


def _sc_esum(src2, ew3):
    return pl.kernel(
        _sc_esum_body,
        out_type=jax.ShapeDtypeStruct((_NC, _NP, 16), jnp.float32),
        mesh=_vec_mesh(),
        compiler_params=_sc_cp,
        scratch_types=[
            pltpu.VMEM((_NBAT, _BB), jnp.int32),
            pltpu.VMEM((_BB, 16), jnp.float32),
            pltpu.VMEM((_BB, 16), jnp.float32),
            pltpu.VMEM((_BB, 16), jnp.float32),
            pltpu.VMEM_SHARED((_NP, 16), jnp.float32),
            pltpu.SemaphoreType.DMA,
            pltpu.SemaphoreType.DMA,
            pltpu.SemaphoreType.DMA,
        ],
    )(src2, ew3)


def _sc_bc_body(src2_hbm, dst2_hbm, ew3_hbm, y12_hbm, out_hbm,
                srcv2, dstv2, crows0, grows0, srows0, crows1, grows1, srows1,
                zbuf, acc, sem, semc0, semg0, semc1, semg1):
    cid = lax.axis_index("c")
    sid = lax.axis_index("s")
    wid = sid * _NC + cid
    brow = wid * _NBAT
    z16 = jnp.zeros((_L,), jnp.float32)

    @pl.loop(0, _BB)
    def _(i):
        zbuf[i, :] = z16

    @pl.loop(0, _CHUNK // _BB)
    def _(k):
        pltpu.sync_copy(zbuf, acc.at[pl.ds(sid * _CHUNK + k * _BB, _BB)])

    pltpu.async_copy(src2_hbm.at[pl.ds(brow, _NBAT)], srcv2, sem).wait()
    pltpu.async_copy(dst2_hbm.at[pl.ds(brow, _NBAT)], dstv2, sem).wait()
    plsc.subcore_barrier()

    @pl.loop(0, _NBAT - 1, step=2)
    def _(b):
        cpc0 = pltpu.async_copy(ew3_hbm.at[brow + b], crows0, semc0)
        cpg0 = pltpu.async_copy(y12_hbm.at[srcv2.at[b]], grows0, semg0)
        cpc1 = pltpu.async_copy(ew3_hbm.at[brow + b + 1], crows1, semc1)
        cpg1 = pltpu.async_copy(y12_hbm.at[srcv2.at[b + 1]], grows1, semg1)
        cpc0.wait()
        cpg0.wait()

        @pl.loop(0, _BB)
        def _(j):
            srows0[j, :] = grows0[j, :] * crows0[j, :]

        pltpu.sync_copy(srows0, acc.at[dstv2.at[b]], add=True)
        cpc1.wait()
        cpg1.wait()

        @pl.loop(0, _BB)
        def _(j):
            srows1[j, :] = grows1[j, :] * crows1[j, :]

        pltpu.sync_copy(srows1, acc.at[dstv2.at[b + 1]], add=True)

    cpcT = pltpu.async_copy(ew3_hbm.at[brow + _NBAT - 1], crows0, semc0)
    cpgT = pltpu.async_copy(y12_hbm.at[srcv2.at[_NBAT - 1]], grows0, semg0)
    cpcT.wait()
    cpgT.wait()

    @pl.loop(0, _BB)
    def _(j):
        srows0[j, :] = grows0[j, :] * crows0[j, :]

    pltpu.sync_copy(srows0, acc.at[dstv2.at[_NBAT - 1]], add=True)

    plsc.subcore_barrier()
    pltpu.sync_copy(acc.at[pl.ds(sid * _CHUNK, _CHUNK)],
                    out_hbm.at[cid, pl.ds(sid * _CHUNK, _CHUNK)])


def _sc_bc(src2, dst2, ew3, y12):
    return pl.kernel(
        _sc_bc_body,
        out_type=jax.ShapeDtypeStruct((_NC, _NP, 16), jnp.float32),
        mesh=_vec_mesh(),
        compiler_params=_sc_cp,
        scratch_types=[
            pltpu.VMEM((_NBAT, _BB), jnp.int32),
            pltpu.VMEM((_NBAT, _BB), jnp.int32),
            pltpu.VMEM((_BB, 16), jnp.float32),
            pltpu.VMEM((_BB, 16), jnp.float32),
            pltpu.VMEM((_BB, 16), jnp.float32),
            pltpu.VMEM((_BB, 16), jnp.float32),
            pltpu.VMEM((_BB, 16), jnp.float32),
            pltpu.VMEM((_BB, 16), jnp.float32),
            pltpu.VMEM((_BB, 16), jnp.float32),
            pltpu.VMEM_SHARED((_NP, 16), jnp.float32),
            pltpu.SemaphoreType.DMA,
            pltpu.SemaphoreType.DMA,
            pltpu.SemaphoreType.DMA,
            pltpu.SemaphoreType.DMA,
            pltpu.SemaphoreType.DMA,
        ],
    )(src2, dst2, ew3, y12)


def _sc_a_body(src2_hbm, dst2_hbm, e2_hbm, s_hbm, inv_hbm, out_hbm,
               srcv2, dstv2, ev2, sv, invv, vtmp, rowbuf, zbuf, acc, sem):
    cid = lax.axis_index("c")
    sid = lax.axis_index("s")
    wid = sid * _NC + cid
    brow = wid * _NBAT
    z16 = jnp.zeros((_L,), jnp.float32)

    @pl.loop(0, _BB)
    def _(i):
        zbuf[i, :] = z16

    @pl.loop(0, _CHUNK // _BB)
    def _(k):
        pltpu.sync_copy(zbuf, acc.at[pl.ds(sid * _CHUNK + k * _BB, _BB)])

    pltpu.async_copy(src2_hbm.at[pl.ds(brow, _NBAT)], srcv2, sem).wait()
    pltpu.async_copy(dst2_hbm.at[pl.ds(brow, _NBAT)], dstv2, sem).wait()
    pltpu.async_copy(e2_hbm.at[pl.ds(brow, _NBAT)], ev2, sem).wait()
    pltpu.async_copy(s_hbm, sv, sem).wait()
    pltpu.async_copy(inv_hbm, invv, sem).wait()
    plsc.subcore_barrier()

    @pl.loop(0, _NBAT)
    def _(b):
        @pl.loop(0, _BB, step=_L)
        def _(j0):
            sidx = srcv2[b, pl.ds(j0, _L)]
            v = plsc.load_gather(sv, [sidx]) * plsc.load_gather(invv, [sidx])
            vtmp[...] = v * ev2[b, pl.ds(j0, _L)]
            for jj in range(_L):
                cst = jnp.full((_L,), jj, jnp.int32)
                rowbuf[j0 + jj, :] = plsc.load_gather(vtmp, [cst])

        pltpu.sync_copy(rowbuf, acc.at[dstv2.at[b]], add=True)

    plsc.subcore_barrier()
    pltpu.sync_copy(acc.at[pl.ds(sid * _CHUNK, _CHUNK)],
                    out_hbm.at[cid, pl.ds(sid * _CHUNK, _CHUNK)])


def _sc_a(src2, dst2, e2, s, inv):
    return pl.kernel(
        _sc_a_body,
        out_type=jax.ShapeDtypeStruct((_NC, _NP, 16), jnp.float32),
        mesh=_vec_mesh(),
        compiler_params=_sc_cp,
        scratch_types=[
            pltpu.VMEM((_NBAT, _BB), jnp.int32),
            pltpu.VMEM((_NBAT, _BB), jnp.int32),
            pltpu.VMEM((_NBAT, _BB), jnp.float32),
            pltpu.VMEM((_NP,), jnp.float32),
            pltpu.VMEM((_NP,), jnp.float32),
            pltpu.VMEM((_L,), jnp.float32),
            pltpu.VMEM((_BB, 16), jnp.float32),
            pltpu.VMEM((_BB, 16), jnp.float32),
            pltpu.VMEM_SHARED((_NP, 16), jnp.float32),
            pltpu.SemaphoreType.DMA,
        ],
    )(src2, dst2, e2, s, inv)


def _tc_ew_body(w_ref, b_ref, ea_ref, out_ref):
    r0 = ea_ref[0, :]
    r1 = ea_ref[1, :]
    r2 = ea_ref[2, :]
    r3 = ea_ref[3, :]
    out_ref[0, :] = jnp.exp(r0 * w_ref[0, 0] + r1 * w_ref[1, 0]
                            + r2 * w_ref[2, 0] + r3 * w_ref[3, 0] + b_ref[0])
    out_ref[1, :] = jnp.exp(r0 * w_ref[0, 1] + r1 * w_ref[1, 1]
                            + r2 * w_ref[2, 1] + r3 * w_ref[3, 1] + b_ref[1])


def _tc_ew(eaT, we, be):
    nb = 10
    be_blk = _E // nb
    return pl.pallas_call(
        _tc_ew_body,
        grid=(nb,),
        in_specs=[
            pl.BlockSpec(memory_space=pltpu.SMEM),
            pl.BlockSpec(memory_space=pltpu.SMEM),
            pl.BlockSpec((4, be_blk), lambda i: (0, i)),
        ],
        out_specs=pl.BlockSpec((2, be_blk), lambda i: (0, i)),
        out_shape=jax.ShapeDtypeStruct((2, _E), jnp.float32),
    )(we, be, eaT)


def _tc_y12_body(part_ref, xg_ref, y12_ref, inv_ref):
    xg = xg_ref[...]
    es1 = part_ref[0][:, 0:8] + part_ref[1][:, 0:8]
    es2 = part_ref[0][:, 8:16] + part_ref[1][:, 8:16]
    inv1 = jnp.where(es1 > 0, 1.0 / es1, 0.0)
    inv2 = jnp.where(es2 > 0, 1.0 / es2, 0.0)
    inv_ref[:, 0:1] = inv1[:, 0:1]
    inv_ref[:, 1:2] = inv2[:, 0:1]
    y12_ref[:, 0:8] = xg * inv1
    y12_ref[:, 8:16] = xg * inv2


def _tc_y12(esum_part, xgp):
    return pl.pallas_call(
        _tc_y12_body,
        out_shape=(
            jax.ShapeDtypeStruct((_NP, 16), jnp.float32),
            jax.ShapeDtypeStruct((_NP, 2), jnp.float32),
        ),
    )(esum_part, xgp)


def _final_dense_kernel(g3_ref, w_ref, b_ref, out_ref):
    g3 = g3_ref[...]
    res = g3 @ w_ref[...] + b_ref[...]
    res = jnp.where(jnp.arange(8)[None, :] == 0, jnp.clip(res, -0.1, 1.0), res)
    out_ref[...] = res


_BLK = 1024
_NBLK = _NP // _BLK


def _lstm_update(G, c):
    i = jax.nn.sigmoid(G[:, 0:16])
    f = jax.nn.sigmoid(G[:, 16:32])
    g = jnp.tanh(G[:, 32:48])
    o = jax.nn.sigmoid(G[:, 48:64])
    c2 = f * c + i * g
    h2 = o * jnp.tanh(c2)
    return h2, c2


def _tc_enc_body(x_ref, xg_ref, xseq_ref,
                 w0_ref, b0_ref, w1_ref, b1_ref,
                 wg0_ref, bg0_ref, wg1_ref, bg1_ref,
                 wl1_ref, bl1_ref, wl2_ref, bl2_ref, wl2b_ref, bl2b_ref,
                 s1_ref, hsf_ref, hsb_ref):
    f32 = jnp.float32
    bf16 = jnp.bfloat16
    x = x_ref[...]
    prof = jnp.maximum(x @ w0_ref[...] + b0_ref[...], 0.0) @ w1_ref[...] + b1_ref[...]
    xg = xg_ref[...]
    geo = jnp.maximum(xg @ wg0_ref[...] + bg0_ref[...], 0.0) @ wg1_ref[...] + bg1_ref[...]

    wl1 = wl1_ref[...]
    bl1 = bl1_ref[...]
    B = _BLK
    z = jnp.zeros((B, 16), f32)

    def step1(t, carry):
        hf, cf, hb, cb = carry
        xtf = xseq_ref[t].astype(bf16)
        xtb = xseq_ref[23 - t].astype(bf16)
        A = jnp.concatenate([xtf, hf.astype(bf16), xtb, hb.astype(bf16)], axis=1)
        G = jnp.dot(A, wl1, preferred_element_type=f32) + bl1
        hf, cf = _lstm_update(G[:, 0:64], cf)
        hb, cb = _lstm_update(G[:, 64:128], cb)
        hsf_ref[t] = hf.astype(bf16)
        hsb_ref[23 - t] = hb.astype(bf16)
        return hf, cf, hb, cb

    lax.fori_loop(0, 24, step1, (z, z, z, z))

    wl2 = wl2_ref[...]
    bl2 = bl2_ref[...]

    def step2a(i, carry):
        h2f, c2f, h2b, c2b = carry
        A = jnp.concatenate([hsf_ref[i], hsb_ref[i], h2f.astype(bf16),
                             hsf_ref[23 - i], hsb_ref[23 - i], h2b.astype(bf16)], axis=1)
        G = jnp.dot(A, wl2, preferred_element_type=f32) + bl2
        h2f, c2f = _lstm_update(G[:, 0:64], c2f)
        h2b, c2b = _lstm_update(G[:, 64:128], c2b)
        return h2f, c2f, h2b, c2b

    h2f, _, h2b, c2b = lax.fori_loop(0, 6, step2a, (z, z, z, z))
    t5f = h2f

    wl2b = wl2b_ref[...]
    bl2b = bl2b_ref[...]

    def step2b(i, carry):
        h2b, c2b = carry
        A = jnp.concatenate([hsf_ref[23 - i], hsb_ref[23 - i], h2b.astype(bf16)], axis=1)
        G = jnp.dot(A, wl2b, preferred_element_type=f32) + bl2b
        return _lstm_update(G, c2b)

    t5b, _ = lax.fori_loop(6, 19, step2b, (h2b, c2b))

    s = (jnp.sum(prof, axis=1, keepdims=True)
         + jnp.sum(geo, axis=1, keepdims=True)
         + jnp.sum(t5f, axis=1, keepdims=True)
         + jnp.sum(t5b, axis=1, keepdims=True))
    s1_ref[...] = s


def _tc_encoder(x_p, xgp, xseq, wlist):
    full = lambda shape: pl.BlockSpec(shape, lambda i: tuple(0 for _ in shape))
    in_specs = [
        pl.BlockSpec((_BLK, 128), lambda i: (i, 0)),
        pl.BlockSpec((_BLK, 8), lambda i: (i, 0)),
        pl.BlockSpec((24, _BLK, 8), lambda i: (0, i, 0)),
    ] + [full(w.shape) for w in wlist]
    return pl.pallas_call(
        _tc_enc_body,
        grid=(_NBLK,),
        in_specs=in_specs,
        out_specs=pl.BlockSpec((_BLK, 1), lambda i: (i, 0)),
        out_shape=jax.ShapeDtypeStruct((_NP, 1), jnp.float32),
        scratch_shapes=[
            pltpu.VMEM((24, _BLK, 16), jnp.bfloat16),
            pltpu.VMEM((24, _BLK, 16), jnp.bfloat16),
        ],
    )(x_p, xgp, xseq, *wlist)


def _tc_ubuild_body(co, apart_ref, bcpart_ref, xg_ref, sc_ref, ucat_ref, g_ref, su_ref):
    A8 = apart_ref[0][:, 0:8] + apart_ref[1][:, 0:8]
    BC = bcpart_ref[0][:, co:co + 8] + bcpart_ref[1][:, co:co + 8]
    zcol = jnp.zeros((_NP, 1), jnp.float32)
    U = jnp.concatenate([BC[:, 0:6], A8[:, 0:1], zcol], axis=1)
    xgs = jnp.concatenate([xg_ref[...][:, 0:6], sc_ref[...], zcol], axis=1)
    ucat_ref[...] = jnp.concatenate([U, xgs], axis=1)
    g_ref[...] = jax.lax.dot_general(U, U, (((0,), (0,)), ((), ())),
                                     preferred_element_type=jnp.float32,
                                     precision=jax.lax.Precision.HIGHEST)
    su_ref[...] = jnp.sum(U, axis=0, keepdims=True)


def _tc_ubuild(co, a_part, bc_part, xgp, scol):
    import functools
    return pl.pallas_call(
        functools.partial(_tc_ubuild_body, co),
        out_shape=(
            jax.ShapeDtypeStruct((_NP, 16), jnp.float32),
            jax.ShapeDtypeStruct((8, 8), jnp.float32),
            jax.ShapeDtypeStruct((1, 8), jnp.float32),
        ),
    )(a_part, bc_part, xgp, scol)


def _tc_dense_body(final, ucat_ref, g_ref, su_ref, wcat_ref, m_ref,
                   bng_ref, bnb_ref, wall_ref, ball_ref, out_ref):
    R = jnp.dot(ucat_ref[...], wcat_ref[...],
                preferred_element_type=jnp.float32,
                precision=jax.lax.Precision.HIGHEST)
    out = R[:, 0:32]
    xt = R[:, 32:64]
    M = m_ref[...]
    ninv = 1.0 / _N
    mean = jnp.dot(su_ref[...], M, precision=jax.lax.Precision.HIGHEST) * ninv
    GM = jnp.dot(g_ref[...], M, precision=jax.lax.Precision.HIGHEST)
    e2 = jnp.sum(M * GM, axis=0, keepdims=True) * ninv
    ve = e2 - mean * mean + 1e-5
    r = jax.lax.rsqrt(ve)
    r = r * (1.5 - 0.5 * ve * r * r)
    outn = (out - mean) * r * bng_ref[...] + bnb_ref[...]
    g2 = jnp.maximum(outn, 0.0) + xt
    if final:
        res = jnp.dot(g2, wall_ref[...], precision=jax.lax.Precision.HIGHEST) + ball_ref[...]
        res = jnp.where(jnp.arange(8)[None, :] == 0, jnp.clip(res, -0.1, 1.0), res)
        out_ref[...] = res
    else:
        out_ref[...] = jnp.sum(g2, axis=1, keepdims=True)


def _tc_dense(final, ucat, G, su, wcat, M, bng, bnb, wall, ball):
    import functools
    oshape = (_NP, 8) if final else (_NP, 1)
    return pl.pallas_call(
        functools.partial(_tc_dense_body, final),
        out_shape=jax.ShapeDtypeStruct(oshape, jnp.float32),
    )(ucat, G, su, wcat, M, bng, bnb, wall, ball)


def kernel(x, x_geo, time_series_profile, edge_attr, params, edge_index):
    p = params
    f32 = jnp.float32
    src = edge_index[0].astype(jnp.int32)
    dst = edge_index[1].astype(jnp.int32)

    we = jnp.stack([p['gnn'][0]['edge_w'][0], p['gnn'][1]['edge_w'][0]], axis=1)
    be = jnp.stack([p['gnn'][0]['edge_b'][0], p['gnn'][1]['edge_b'][0]])
    eaT = edge_attr.T

    xgp = jnp.zeros((_NP, 8), f32)
    xgp = xgp.at[:_N, :5].set(x_geo).at[:_N, 5].set(1.0)

    e12 = _tc_ew(eaT, we, be)
    e1 = e12[0]
    e2 = e12[1]
    ewide = jnp.repeat(e12.T, 8, axis=1)

    src2 = src.reshape(_E // _BB, _BB)
    dst2 = dst.reshape(_E // _BB, _BB)
    ew3 = ewide.reshape(_E // _BB, _BB, 16)
    e1b2 = e1.reshape(_E // _BB, _BB)
    e2b2 = e2.reshape(_E // _BB, _BB)

    esum_part = _sc_esum(src2, ew3)

    y12, inv12 = _tc_y12(esum_part, xgp)
    inv1 = inv12[:, 0]
    inv2 = inv12[:, 1]

    bc_part = _sc_bc(src2, dst2, ew3, y12)

    l1, l2 = p['lstm']

    def wcat(lp, in_d):
        W = jnp.zeros((2 * (in_d + 16), 128), f32)
        W = W.at[0:in_d, 0:64].set(lp['wih_f'].T)
        W = W.at[in_d:in_d + 16, 0:64].set(lp['whh_f'].T)
        W = W.at[in_d + 16:2 * in_d + 16, 64:128].set(lp['wih_b'].T)
        W = W.at[2 * in_d + 16:, 64:128].set(lp['whh_b'].T)
        b = jnp.concatenate([lp['bih_f'] + lp['bhh_f'], lp['bih_b'] + lp['bhh_b']])[None]
        return W.astype(jnp.bfloat16), b

    wl1, bl1 = wcat(l1, 8)
    wl2, bl2 = wcat(l2, 32)
    wl2b = jnp.concatenate([l2['wih_b'].T, l2['whh_b'].T], axis=0).astype(jnp.bfloat16)
    bl2b = (l2['bih_b'] + l2['bhh_b'])[None]

    wg0 = jnp.zeros((8, 32), f32).at[:5, :].set(p['geo_w0'].T)
    wlist = [
        p['mlp_w0'].T, p['mlp_b0'][None], p['mlp_w1'].T, p['mlp_b1'][None],
        wg0, p['geo_b0'][None], p['geo_w1'].T, p['geo_b1'][None],
        wl1, bl1, wl2, bl2, wl2b, bl2b,
    ]
    x_p = jnp.pad(x, ((0, _NP - _N), (0, 0)))
    xseq = jnp.pad(jnp.transpose(time_series_profile, (2, 0, 1)),
                   ((0, 0), (0, _NP - _N), (0, 0)))
    s1col = _tc_encoder(x_p, xgp, xseq, wlist)

    def mk_M(gp):
        fsum = gp['feat_w'].sum(1)
        M = jnp.zeros((8, 32), f32)
        M = M.at[0:5, :].set(gp['beta_w'].T)
        M = M.at[5, :].set(gp['beta_b'])
        M = M.at[6, :].set(fsum)
        bwaug = jnp.zeros((8, 32), f32)
        bwaug = bwaug.at[0:5, :].set(gp['beta_w'].T)
        bwaug = bwaug.at[5, :].set(gp['beta_b'])
        bwaug = bwaug.at[6, :].set(fsum)
        wcat = jnp.zeros((16, 64), f32)
        wcat = wcat.at[0:8, 0:32].set(M)
        wcat = wcat.at[8:16, 32:64].set(bwaug)
        return M, wcat, gp['bn_g'][None], gp['bn_b'][None]

    M1, wcat1, bng1, bnb1 = mk_M(p['gnn'][0])
    M2, wcat2, bng2, bnb2 = mk_M(p['gnn'][1])
    w_all = jnp.zeros((32, 8), f32).at[:, 0].set(p['out_w'][0]).at[:, 1:3].set(p['sp_w'].T)
    b_all = jnp.zeros((1, 8), f32).at[0, 0].set(p['out_b'][0]).at[0, 1:3].set(p['sp_b'])

    s1flat = s1col[:, 0]
    a1_part = _sc_a(src2, dst2, e1b2, s1flat, inv1)

    ucat1, G1, su1 = _tc_ubuild(0, a1_part, bc_part, xgp, s1col)
    s2col = _tc_dense(False, ucat1, G1, su1, wcat1, M1, bng1, bnb1, w_all, b_all)

    a2_part = _sc_a(src2, dst2, e2b2, s2col[:, 0], inv2)

    ucat2, G2, su2 = _tc_ubuild(8, a2_part, bc_part, xgp, s2col)
    res = _tc_dense(True, ucat2, G2, su2, wcat2, M2, bng2, bnb2, w_all, b_all)
    return (res[:_N, 0:1], res[:_N, 1:3])
